# baseline (speedup 1.0000x reference)
import math

import jax
import jax.numpy as jnp
from jax import lax
from jax.experimental import pallas as pl
from jax.experimental.pallas import tpu as pltpu
from jax.experimental.pallas import tpu_sc as plsc

EMB = 64
SCALE = math.sqrt(EMB)
NW = 32
K1_BLOCK = 2048
LANES = 128


def _scale_dup(table_t):
    vocab = table_t.shape[1]

    def body(x_ref, o_ref):
        x = x_ref[...].T * SCALE
        o_ref[...] = jnp.concatenate([x, x], axis=-1)

    grid = (vocab + K1_BLOCK - 1) // K1_BLOCK
    return pl.pallas_call(
        body,
        grid=(grid,),
        in_specs=[pl.BlockSpec((EMB, K1_BLOCK), lambda i: (0, i))],
        out_specs=pl.BlockSpec((K1_BLOCK, 2 * EMB), lambda i: (i, 0)),
        out_shape=jax.ShapeDtypeStruct((vocab, 2 * EMB), jnp.float32),
    )(table_t)


def kernel(tokens, embedding_weight):
    n_rows, n_cols = tokens.shape
    idx_t = tokens.T.astype(jnp.int32)

    mesh = plsc.VectorSubcoreMesh(core_axis_name="core", subcore_axis_name="subcore")

    @jax.jit
    def run(table_t, indices_t):
        tabled = _scale_dup(table_t)

        @pl.kernel(
            out_type=jax.ShapeDtypeStruct((n_cols, EMB, n_rows), jnp.float32),
            mesh=mesh,
            scratch_types=[
                pltpu.VMEM((n_cols, LANES), jnp.int32),
                pltpu.VMEM((LANES, 2 * EMB), jnp.float32),
                pltpu.VMEM((LANES, 2 * EMB), jnp.float32),
                pltpu.VMEM((EMB, LANES), jnp.float32),
                pltpu.VMEM((EMB, LANES), jnp.float32),
                pltpu.SemaphoreType.DMA,
                pltpu.SemaphoreType.DMA((2,)),
                pltpu.SemaphoreType.DMA((2,)),
            ],
            compiler_params=pltpu.CompilerParams(needs_layout_passes=False),
        )
        def k(x_hbm, i_hbm, o_hbm, idx_v, g_v0, g_v1, o_b0, o_b1, isem, gsems, osems):
            wid = lax.axis_index("subcore") * 2 + lax.axis_index("core")
            b0 = wid * LANES

            for r8 in range(0, n_cols, 8):
                pltpu.async_copy(
                    i_hbm.at[pl.ds(r8, 8), pl.ds(b0, LANES)],
                    idx_v.at[pl.ds(r8, 8)],
                    isem,
                ).wait()

            iotas = [lax.iota(jnp.int32, 16) + (kk * 16) for kk in range(8)]

            def fire(c, b):
                return pltpu.async_copy(
                    x_hbm.at[idx_v.at[c]],
                    g_v0 if b == 0 else g_v1,
                    gsems.at[b],
                )

            def transpose(gv, o_b):
                for f in range(EMB):
                    colv = jnp.full((16,), f, jnp.int32)
                    for kk in range(8):
                        vals = plsc.load_gather(gv, [iotas[kk], colv])
                        o_b[f, pl.ds(kk * 16, 16)] = vals

            fire(0, 0)
            fire(1, 1)

            @pl.loop(0, n_cols // 2)
            def _(l):
                for b in range(2):
                    c = l * 2 + b
                    gv = g_v0 if b == 0 else g_v1
                    o_b = o_b0 if b == 0 else o_b1
                    pltpu.make_async_copy(
                        x_hbm.at[idx_v.at[c]], gv, gsems.at[b]
                    ).wait()

                    @pl.when(l > 0)
                    def _():
                        pltpu.make_async_copy(
                            o_b, o_hbm.at[c, :, pl.ds(b0, LANES)], osems.at[b]
                        ).wait()

                    transpose(gv, o_b)

                    @pl.when(l < n_cols // 2 - 1)
                    def _():
                        fire(c + 2, b)

                    pltpu.async_copy(
                        o_b, o_hbm.at[c, :, pl.ds(b0, LANES)], osems.at[b]
                    )

            for b in range(2):
                pltpu.make_async_copy(
                    o_b0 if b == 0 else o_b1,
                    o_hbm.at[0, :, pl.ds(b0, LANES)],
                    osems.at[b],
                ).wait()

        return k(tabled, indices_t)

    out_t = run(embedding_weight.T, idx_t)
    return jnp.transpose(out_t, (2, 0, 1))


# speedup vs baseline: 1.9885x; 1.9885x over previous
import math

import jax
import jax.numpy as jnp
from jax import lax
from jax.experimental import pallas as pl
from jax.experimental.pallas import tpu as pltpu
from jax.experimental.pallas import tpu_sc as plsc

EMB = 64
SCALE = math.sqrt(EMB)
TOK_PER_WIN = 200
WPG = 16
NW = 32
TOK_PER_W = 819200 // NW
WINS_PER_W = TOK_PER_W // TOK_PER_WIN
GROUPS = WINS_PER_W // WPG
GROUP_ROWS = WPG * TOK_PER_WIN // 128
K1_BLOCK = 8192


def _window_pieces(q):
    lo, hi = q * TOK_PER_WIN, (q + 1) * TOK_PER_WIN
    bounds = [lo] + [b for b in range((lo // 128 + 1) * 128, hi, 128)] + [hi]
    return [(a, b - a) for a, b in zip(bounds[:-1], bounds[1:])]


def _scale_dup(table_t):
    vocab = table_t.shape[1]

    def body(x_ref, o_ref):
        x = x_ref[...].T * SCALE
        o_ref[...] = jnp.concatenate([x, x], axis=-1)

    grid = (vocab + K1_BLOCK - 1) // K1_BLOCK
    return pl.pallas_call(
        body,
        grid=(grid,),
        in_specs=[pl.BlockSpec((EMB, K1_BLOCK), lambda i: (0, i))],
        out_specs=pl.BlockSpec((K1_BLOCK, 2 * EMB), lambda i: (i, 0)),
        out_shape=jax.ShapeDtypeStruct((vocab, 2 * EMB), jnp.float32),
    )(table_t)


def kernel(tokens, embedding_weight):
    n_rows, n_cols = tokens.shape
    n_tok = n_rows * n_cols
    idx = tokens.reshape(n_tok // 128, 128).astype(jnp.int32)

    mesh = plsc.VectorSubcoreMesh(core_axis_name="core", subcore_axis_name="subcore")

    @jax.jit
    def run(table_t, indices):
        tabled = _scale_dup(table_t)

        @pl.kernel(
            out_type=jax.ShapeDtypeStruct((n_rows, n_cols, EMB), jnp.float32),
            mesh=mesh,
            scratch_types=[
                pltpu.VMEM((TOK_PER_W // 128, 128), jnp.int32),
                pltpu.VMEM((128, 2 * EMB), jnp.float32),
                pltpu.VMEM((128, 2 * EMB), jnp.float32),
                pltpu.VMEM((TOK_PER_WIN, EMB), jnp.float32),
                pltpu.VMEM((TOK_PER_WIN, EMB), jnp.float32),
                pltpu.SemaphoreType.DMA,
                pltpu.SemaphoreType.DMA((2,)),
                pltpu.SemaphoreType.DMA((2,)),
            ],
        )
        def k(x_hbm, i_hbm, o_hbm, idx_v, g_v0, g_v1, o_v0, o_v1, isem, gsems, osems):
            o64 = o_hbm.reshape(n_tok, EMB)
            wid = lax.axis_index("subcore") * 2 + lax.axis_index("core")
            irow0 = wid * (TOK_PER_W // 128)

            for r8 in range(0, TOK_PER_W // 128, 8):
                pltpu.async_copy(
                    i_hbm.at[pl.ds(irow0 + r8, 8)],
                    idx_v.at[pl.ds(r8, 8)],
                    isem,
                ).wait()

            def select(cnt, off, gv, o_v):
                @pl.loop(0, cnt)
                def _(r):
                    src = gv.at[r]
                    dst = o_v.at[off + r]
                    for c in range(EMB // 16):
                        dst[pl.ds(c * 16, 16)] = src[pl.ds(c * 16, 16)]

            @pl.loop(0, GROUPS)
            def _(grp):
                grow0 = grp * GROUP_ROWS
                inflight = []

                def fire(flat, cnt, j, o_v, off):
                    gv = g_v0 if j % 2 == 0 else g_v1
                    if len(inflight) >= 2:
                        cp, cnt_p, off_p, gv_p, ov_p = inflight.pop(0)
                        cp.wait()
                        select(cnt_p, off_p, gv_p, ov_p)
                    cp = pltpu.async_copy(
                        x_hbm.at[idx_v.at[grow0 + flat // 128,
                                          pl.ds(flat % 128, cnt)]],
                        gv.at[pl.ds(0, cnt)],
                        gsems.at[j % 2],
                    )
                    inflight.append((cp, cnt, off, gv, o_v))

                def drain():
                    while inflight:
                        cp, cnt_p, off_p, gv_p, ov_p = inflight.pop(0)
                        cp.wait()
                        select(cnt_p, off_p, gv_p, ov_p)

                j = 0
                for q in range(WPG):
                    o_v = o_v0 if q % 2 == 0 else o_v1
                    osem = osems.at[q % 2]
                    win = grp * WPG + q
                    tok0 = wid * TOK_PER_W + win * TOK_PER_WIN

                    if q >= 2:
                        pltpu.make_async_copy(
                            o_v, o64.at[pl.ds(tok0, TOK_PER_WIN)], osem
                        ).wait()
                    else:
                        @pl.when(grp > 0)
                        def _():
                            pltpu.make_async_copy(
                                o_v, o64.at[pl.ds(tok0, TOK_PER_WIN)], osem
                            ).wait()

                    for flat, cnt in _window_pieces(q):
                        fire(flat, cnt, j, o_v, flat - q * TOK_PER_WIN)
                        j += 1
                    drain()

                    pltpu.async_copy(o_v, o64.at[pl.ds(tok0, TOK_PER_WIN)], osem)

            for b in range(2):
                pltpu.make_async_copy(
                    o_v0 if b == 0 else o_v1,
                    o64.at[pl.ds(0, TOK_PER_WIN)],
                    osems.at[b],
                ).wait()

        return k(tabled, indices)

    return run(embedding_weight.T, idx)


# speedup vs baseline: 2.0656x; 1.0388x over previous
import math

import jax
import jax.numpy as jnp
from jax import lax
from jax.experimental import pallas as pl
from jax.experimental.pallas import tpu as pltpu
from jax.experimental.pallas import tpu_sc as plsc

EMB = 64
SCALE = math.sqrt(EMB)
TOK_PER_WIN = 200
WPG = 16
NW = 32
TOK_PER_W = 819200 // NW
WINS_PER_W = TOK_PER_W // TOK_PER_WIN
GROUPS = WINS_PER_W // WPG
GROUP_ROWS = WPG * TOK_PER_WIN // 128
K1_BLOCK = 16384


def _window_pieces(q):
    lo, hi = q * TOK_PER_WIN, (q + 1) * TOK_PER_WIN
    bounds = [lo] + [b for b in range((lo // 128 + 1) * 128, hi, 128)] + [hi]
    return [(a, b - a) for a, b in zip(bounds[:-1], bounds[1:])]


def _scale_dup(table_t):
    vocab = table_t.shape[1]

    def body(x_ref, o_ref):
        x = x_ref[...].T * SCALE
        o_ref[...] = jnp.concatenate([x, x], axis=-1)

    grid = (vocab + K1_BLOCK - 1) // K1_BLOCK
    return pl.pallas_call(
        body,
        grid=(grid,),
        in_specs=[pl.BlockSpec((EMB, K1_BLOCK), lambda i: (0, i))],
        out_specs=pl.BlockSpec((K1_BLOCK, 2 * EMB), lambda i: (i, 0)),
        out_shape=jax.ShapeDtypeStruct((vocab, 2 * EMB), jnp.float32),
    )(table_t)


def kernel(tokens, embedding_weight):
    n_rows, n_cols = tokens.shape
    n_tok = n_rows * n_cols
    idx = tokens.reshape(n_tok // 128, 128).astype(jnp.int32)

    mesh = plsc.VectorSubcoreMesh(core_axis_name="core", subcore_axis_name="subcore")

    @jax.jit
    def run(table_t, indices):
        tabled = _scale_dup(table_t)

        @pl.kernel(
            out_type=jax.ShapeDtypeStruct((n_rows, n_cols, EMB), jnp.float32),
            mesh=mesh,
            scratch_types=[
                pltpu.VMEM((TOK_PER_W // 128, 128), jnp.int32),
                pltpu.VMEM((128, 2 * EMB), jnp.float32),
                pltpu.VMEM((128, 2 * EMB), jnp.float32),
                pltpu.VMEM((TOK_PER_WIN, EMB), jnp.float32),
                pltpu.VMEM((TOK_PER_WIN, EMB), jnp.float32),
                pltpu.SemaphoreType.DMA,
                pltpu.SemaphoreType.DMA((2,)),
                pltpu.SemaphoreType.DMA((2,)),
            ],
        )
        def k(x_hbm, i_hbm, o_hbm, idx_v, g_v0, g_v1, o_v0, o_v1, isem, gsems, osems):
            o64 = o_hbm.reshape(n_tok, EMB)
            wid = lax.axis_index("subcore") * 2 + lax.axis_index("core")
            irow0 = wid * (TOK_PER_W // 128)

            for r8 in range(0, TOK_PER_W // 128, 8):
                pltpu.async_copy(
                    i_hbm.at[pl.ds(irow0 + r8, 8)],
                    idx_v.at[pl.ds(r8, 8)],
                    isem,
                ).wait()

            def select(cnt, off, gv, o_v):
                @pl.loop(0, cnt)
                def _(r):
                    src = gv.at[r]
                    dst = o_v.at[off + r]
                    for c in range(EMB // 16):
                        dst[pl.ds(c * 16, 16)] = src[pl.ds(c * 16, 16)]

            @pl.loop(0, GROUPS)
            def _(grp):
                grow0 = grp * GROUP_ROWS
                inflight = []

                def fire(flat, cnt, j, o_v, off):
                    gv = g_v0 if j % 2 == 0 else g_v1
                    if len(inflight) >= 2:
                        cp, cnt_p, off_p, gv_p, ov_p = inflight.pop(0)
                        cp.wait()
                        select(cnt_p, off_p, gv_p, ov_p)
                    cp = pltpu.async_copy(
                        x_hbm.at[idx_v.at[grow0 + flat // 128,
                                          pl.ds(flat % 128, cnt)]],
                        gv.at[pl.ds(0, cnt)],
                        gsems.at[j % 2],
                    )
                    inflight.append((cp, cnt, off, gv, o_v))

                def drain():
                    while inflight:
                        cp, cnt_p, off_p, gv_p, ov_p = inflight.pop(0)
                        cp.wait()
                        select(cnt_p, off_p, gv_p, ov_p)

                j = 0
                for q in range(WPG):
                    o_v = o_v0 if q % 2 == 0 else o_v1
                    osem = osems.at[q % 2]
                    win = grp * WPG + q
                    tok0 = wid * TOK_PER_W + win * TOK_PER_WIN

                    if q >= 2:
                        pltpu.make_async_copy(
                            o_v, o64.at[pl.ds(tok0, TOK_PER_WIN)], osem
                        ).wait()
                    else:
                        @pl.when(grp > 0)
                        def _():
                            pltpu.make_async_copy(
                                o_v, o64.at[pl.ds(tok0, TOK_PER_WIN)], osem
                            ).wait()

                    for flat, cnt in _window_pieces(q):
                        fire(flat, cnt, j, o_v, flat - q * TOK_PER_WIN)
                        j += 1
                    drain()

                    pltpu.async_copy(o_v, o64.at[pl.ds(tok0, TOK_PER_WIN)], osem)

            for b in range(2):
                pltpu.make_async_copy(
                    o_v0 if b == 0 else o_v1,
                    o64.at[pl.ds(0, TOK_PER_WIN)],
                    osems.at[b],
                ).wait()

        return k(tabled, indices)

    return run(embedding_weight.T, idx)
